# baseline (speedup 1.0000x reference)
import functools

import jax
import jax.numpy as jnp
from jax import lax
from jax.experimental import pallas as pl
from jax.experimental.pallas import tpu as pltpu
from jax.experimental.pallas import tpu_sc as plsc

_MAX_DEG = 32
_DEPTH = 2
_NEIGH = (16, 8)
_LANES = 16


@functools.cache
def _sampler_kernel(batch):
    info = plsc.get_sparse_core_info()
    nc, ns = info.num_cores, info.num_subcores
    nw = nc * ns
    seeds_w = batch // nw
    assert seeds_w * nw == batch
    h1_w = seeds_w * _NEIGH[0]
    h2_chunk = seeds_w
    n_sub = h1_w // h2_chunk
    h2_out_chunk = h2_chunk * _NEIGH[1]
    k1 = batch * _NEIGH[0]
    k2 = k1 * _NEIGH[1]
    mesh = plsc.VectorSubcoreMesh(core_axis_name="c", subcore_axis_name="s")

    odt = lambda n, dt: jax.ShapeDtypeStruct((n,), dt)

    @functools.partial(
        pl.kernel,
        mesh=mesh,
        compiler_params=pltpu.CompilerParams(
            needs_layout_passes=False, use_tc_tiling_on_sc=False),
        out_type=(
            odt(k1, jnp.int32), odt(k1, jnp.float32),
            odt(k1, jnp.int32), odt(k1, jnp.float32),
            jax.ShapeDtypeStruct((k1, _NEIGH[1]), jnp.int32),
            jax.ShapeDtypeStruct((k1, _NEIGH[1]), jnp.float32),
            jax.ShapeDtypeStruct((k1, _NEIGH[1]), jnp.int32),
            jax.ShapeDtypeStruct((k1, _NEIGH[1]), jnp.float32),
        ),
        scratch_types=[
            pltpu.VMEM((seeds_w,), jnp.int32),
            pltpu.VMEM((h2_chunk, _MAX_DEG), jnp.int32),
            pltpu.VMEM((h2_chunk, _MAX_DEG), jnp.float32),
            pltpu.VMEM((h2_chunk, _MAX_DEG), jnp.int32),
            pltpu.VMEM((h2_chunk, _MAX_DEG), jnp.float32),
            pltpu.VMEM((h1_w,), jnp.int32),
            pltpu.VMEM((h1_w,), jnp.int32),
            pltpu.VMEM((h1_w,), jnp.float32),
            pltpu.VMEM((h2_out_chunk,), jnp.int32),
            pltpu.VMEM((h2_out_chunk,), jnp.int32),
            pltpu.VMEM((h2_chunk, _NEIGH[1]), jnp.int32),
            pltpu.VMEM((h2_chunk, _NEIGH[1]), jnp.float32),
            pltpu.SemaphoreType.DMA,
            pltpu.SemaphoreType.DMA,
            pltpu.SemaphoreType.DMA,
        ],
    )
    def sample(q_hbm, u_hbm, adj_hbm, edge_hbm,
               li0q_hbm, li0u_hbm, li1q_hbm, li1u_hbm,
               qn0_hbm, qe0_hbm, un0_hbm, ue0_hbm,
               qn1_hbm, qe1_hbm, un1_hbm, ue1_hbm,
               seeds_v, arow0_v, erow0_v, arow1_v, erow1_v,
               li1_v, n0_v, e0_v, li2a_v, li2b_v, n1_v, e1_v,
               sem0, sem1, sem2):
        wid = lax.axis_index("s") * nc + lax.axis_index("c")
        arow = (arow0_v, arow1_v)
        erow = (erow0_v, erow1_v)
        li2 = (li2a_v, li2b_v)
        sems = (sem0, sem1)

        lane = lax.iota(jnp.int32, _LANES)
        orow_off = lax.shift_right_logical(lane, 3)
        ocol = jnp.bitwise_and(lane, _NEIGH[1] - 1)

        def select(li_ref, n_ref, e_ref, nvregs, buf):
            def body(v, carry):
                off = v * _LANES
                li = li_ref[pl.ds(off, _LANES)]
                row = lax.shift_right_logical(li, 5)
                col = jnp.bitwise_and(li, _MAX_DEG - 1)
                n_ref[pl.ds(off, _LANES)] = plsc.load_gather(
                    arow[buf], [row, col])
                e_ref[pl.ds(off, _LANES)] = plsc.load_gather(
                    erow[buf], [row, col])
                return carry
            lax.fori_loop(0, nvregs, body, 0)

        def select2(li_ref, n_ref, e_ref, nvregs, buf):
            def body(v, carry):
                li = li_ref[pl.ds(v * _LANES, _LANES)]
                row = lax.shift_right_logical(li, 5)
                col = jnp.bitwise_and(li, _MAX_DEG - 1)
                out_row = orow_off + v * (_LANES // _NEIGH[1])
                plsc.store_scatter(
                    n_ref, [out_row, ocol],
                    plsc.load_gather(arow[buf], [row, col]))
                plsc.store_scatter(
                    e_ref, [out_row, ocol],
                    plsc.load_gather(erow[buf], [row, col]))
                return carry
            lax.fori_loop(0, nvregs, body, 0)

        def branch(seed_hbm, li0_hbm, li1_hbm,
                   n0_hbm, e0_hbm, n1_hbm, e1_hbm):
            sbase = wid * seeds_w
            pltpu.sync_copy(seed_hbm.at[pl.ds(sbase, seeds_w)], seeds_v)
            cp_a = pltpu.async_copy(adj_hbm.at[seeds_v], arow0_v, sem0)
            cp_e = pltpu.async_copy(edge_hbm.at[seeds_v], erow0_v, sem0)
            pltpu.sync_copy(li0_hbm.at[pl.ds(wid * h1_w, h1_w)], li1_v)
            cp_a.wait()
            cp_e.wait()
            select(li1_v, n0_v, e0_v, h1_w // _LANES, 0)
            cp_n0 = pltpu.async_copy(
                n0_v, n0_hbm.at[pl.ds(wid * h1_w, h1_w)], sem2)
            cp_e0 = pltpu.async_copy(
                e0_v, e0_hbm.at[pl.ds(wid * h1_w, h1_w)], sem2)
            obase0 = wid * (h1_w * _NEIGH[1])

            def start(c, buf):
                cur = n0_v.at[pl.ds(c * h2_chunk, h2_chunk)]
                cpa = pltpu.async_copy(adj_hbm.at[cur], arow[buf], sems[buf])
                cpe = pltpu.async_copy(edge_hbm.at[cur], erow[buf], sems[buf])
                cpl = pltpu.async_copy(
                    li1_hbm.at[pl.ds(obase0 + c * h2_out_chunk, h2_out_chunk)],
                    li2[buf], sems[buf])
                return cpa, cpe, cpl

            pend = start(0, 0)
            for c in range(n_sub):
                buf = c % 2
                nxt = start(c + 1, 1 - buf) if c + 1 < n_sub else None
                for cp in pend:
                    cp.wait()
                select2(li2[buf], n1_v, e1_v, h2_out_chunk // _LANES, buf)
                row0 = wid * h1_w + c * h2_chunk
                pltpu.sync_copy(n1_v, n1_hbm.at[pl.ds(row0, h2_chunk)])
                pltpu.sync_copy(e1_v, e1_hbm.at[pl.ds(row0, h2_chunk)])
                pend = nxt
            cp_n0.wait()
            cp_e0.wait()

        branch(q_hbm, li0q_hbm, li1q_hbm, qn0_hbm, qe0_hbm, qn1_hbm, qe1_hbm)
        branch(u_hbm, li0u_hbm, li1u_hbm, un0_hbm, ue0_hbm, un1_hbm, ue1_hbm)

    return sample


def _sample_cols(item_key, batch):
    key = item_key
    cols = []
    m = batch
    for i in range(_DEPTH):
        key, sub = jax.random.split(key)
        cols.append(jax.random.randint(sub, (m, _NEIGH[i]), 0, _MAX_DEG))
        m = m * _NEIGH[i]
    return cols


@functools.cache
def _precomputed_lidx(batch, chunk):
    import numpy as np
    with jax.ensure_compile_time_eval():
        kq, ku = jax.random.split(jax.random.key(42))
        cq = _sample_cols(kq, batch)
        cu = _sample_cols(ku, batch)
        out = tuple(
            np.asarray(_local_idx(c, chunk))
            for c in (cq[0], cu[0], cq[1], cu[1]))
    return out


def _local_idx(cols, chunk):
    m = cols.shape[0]
    local_row = (jnp.arange(m, dtype=jnp.int32) % chunk)[:, None]
    return (local_row * _MAX_DEG + cols).reshape(-1)


def kernel(question, answer_edge, user, adj, adj_edge):
    del answer_edge
    batch = question.shape[0]
    info = plsc.get_sparse_core_info()
    chunk = batch // (info.num_cores * info.num_subcores)
    li0q, li0u, li1q, li1u = _precomputed_lidx(batch, chunk)

    qn0, qe0, un0, ue0, qn1, qe1, un1, ue1 = _sampler_kernel(batch)(
        question, user, adj, adj_edge, li0q, li0u, li1q, li1u)

    m1 = batch
    return (qn0.reshape(m1, _NEIGH[0]), qn1,
            qe0.reshape(m1, _NEIGH[0]), qe1,
            un0.reshape(m1, _NEIGH[0]), un1,
            ue0.reshape(m1, _NEIGH[0]), ue1)


# speedup vs baseline: 1.0091x; 1.0091x over previous
import functools

import jax
import jax.numpy as jnp
from jax import lax
from jax.experimental import pallas as pl
from jax.experimental.pallas import tpu as pltpu
from jax.experimental.pallas import tpu_sc as plsc

_MAX_DEG = 32
_DEPTH = 2
_NEIGH = (16, 8)
_LANES = 16


@functools.cache
def _sampler_kernel(batch):
    info = plsc.get_sparse_core_info()
    nc, ns = info.num_cores, info.num_subcores
    nw = nc * ns
    seeds_w = batch // nw
    assert seeds_w * nw == batch
    h1_w = seeds_w * _NEIGH[0]
    h2_chunk = seeds_w
    n_sub = h1_w // h2_chunk
    h2_out_chunk = h2_chunk * _NEIGH[1]
    k1 = batch * _NEIGH[0]
    k2 = k1 * _NEIGH[1]
    mesh = plsc.VectorSubcoreMesh(core_axis_name="c", subcore_axis_name="s")

    odt = lambda n, dt: jax.ShapeDtypeStruct((n,), dt)

    @functools.partial(
        pl.kernel,
        mesh=mesh,
        compiler_params=pltpu.CompilerParams(
            needs_layout_passes=False, use_tc_tiling_on_sc=False),
        out_type=(
            odt(k1, jnp.int32), odt(k1, jnp.float32),
            odt(k1, jnp.int32), odt(k1, jnp.float32),
            odt(k2, jnp.int32), odt(k2, jnp.float32),
            odt(k2, jnp.int32), odt(k2, jnp.float32),
        ),
        scratch_types=[
            pltpu.VMEM((seeds_w,), jnp.int32),
            pltpu.VMEM((h2_chunk, _MAX_DEG), jnp.int32),
            pltpu.VMEM((h2_chunk, _MAX_DEG), jnp.float32),
            pltpu.VMEM((h2_chunk, _MAX_DEG), jnp.int32),
            pltpu.VMEM((h2_chunk, _MAX_DEG), jnp.float32),
            pltpu.VMEM((h1_w,), jnp.int32),
            pltpu.VMEM((h1_w,), jnp.int32),
            pltpu.VMEM((h1_w,), jnp.float32),
            pltpu.VMEM((h2_out_chunk,), jnp.int32),
            pltpu.VMEM((h2_out_chunk,), jnp.int32),
            pltpu.VMEM((h2_out_chunk,), jnp.int32),
            pltpu.VMEM((h2_out_chunk,), jnp.float32),
            pltpu.SemaphoreType.DMA,
            pltpu.SemaphoreType.DMA,
            pltpu.SemaphoreType.DMA,
        ],
    )
    def sample(q_hbm, u_hbm, adj_hbm, edge_hbm,
               li0q_hbm, li0u_hbm, li1q_hbm, li1u_hbm,
               qn0_hbm, qe0_hbm, un0_hbm, ue0_hbm,
               qn1_hbm, qe1_hbm, un1_hbm, ue1_hbm,
               seeds_v, arow0_v, erow0_v, arow1_v, erow1_v,
               li1_v, n0_v, e0_v, li2a_v, li2b_v, n1_v, e1_v,
               sem0, sem1, sem2):
        wid = lax.axis_index("s") * nc + lax.axis_index("c")
        arow = (arow0_v, arow1_v)
        erow = (erow0_v, erow1_v)
        li2 = (li2a_v, li2b_v)
        sems = (sem0, sem1)

        def select(li_ref, n_ref, e_ref, nvregs, buf):
            def body(v, carry):
                off = v * _LANES
                li = li_ref[pl.ds(off, _LANES)]
                row = lax.shift_right_logical(li, 5)
                col = jnp.bitwise_and(li, _MAX_DEG - 1)
                n_ref[pl.ds(off, _LANES)] = plsc.load_gather(
                    arow[buf], [row, col])
                e_ref[pl.ds(off, _LANES)] = plsc.load_gather(
                    erow[buf], [row, col])
                return carry
            lax.fori_loop(0, nvregs, body, 0)

        def branch(seed_hbm, li0_hbm, li1_hbm,
                   n0_hbm, e0_hbm, n1_hbm, e1_hbm):
            sbase = wid * seeds_w
            pltpu.sync_copy(seed_hbm.at[pl.ds(sbase, seeds_w)], seeds_v)
            cp_a = pltpu.async_copy(adj_hbm.at[seeds_v], arow0_v, sem0)
            cp_e = pltpu.async_copy(edge_hbm.at[seeds_v], erow0_v, sem0)
            pltpu.sync_copy(li0_hbm.at[pl.ds(wid * h1_w, h1_w)], li1_v)
            cp_a.wait()
            cp_e.wait()
            select(li1_v, n0_v, e0_v, h1_w // _LANES, 0)
            cp_n0 = pltpu.async_copy(
                n0_v, n0_hbm.at[pl.ds(wid * h1_w, h1_w)], sem2)
            cp_e0 = pltpu.async_copy(
                e0_v, e0_hbm.at[pl.ds(wid * h1_w, h1_w)], sem2)
            obase0 = wid * (h1_w * _NEIGH[1])

            def start(c, buf):
                cur = n0_v.at[pl.ds(c * h2_chunk, h2_chunk)]
                cpa = pltpu.async_copy(adj_hbm.at[cur], arow[buf], sems[buf])
                cpe = pltpu.async_copy(edge_hbm.at[cur], erow[buf], sems[buf])
                cpl = pltpu.async_copy(
                    li1_hbm.at[pl.ds(obase0 + c * h2_out_chunk, h2_out_chunk)],
                    li2[buf], sems[buf])
                return cpa, cpe, cpl

            pend = start(0, 0)
            for c in range(n_sub):
                buf = c % 2
                nxt = start(c + 1, 1 - buf) if c + 1 < n_sub else None
                for cp in pend:
                    cp.wait()
                select(li2[buf], n1_v, e1_v, h2_out_chunk // _LANES, buf)
                obase = obase0 + c * h2_out_chunk
                pltpu.sync_copy(n1_v, n1_hbm.at[pl.ds(obase, h2_out_chunk)])
                pltpu.sync_copy(e1_v, e1_hbm.at[pl.ds(obase, h2_out_chunk)])
                pend = nxt
            cp_n0.wait()
            cp_e0.wait()

        branch(q_hbm, li0q_hbm, li1q_hbm, qn0_hbm, qe0_hbm, qn1_hbm, qe1_hbm)
        branch(u_hbm, li0u_hbm, li1u_hbm, un0_hbm, ue0_hbm, un1_hbm, ue1_hbm)

    return sample


def _sample_cols(item_key, batch):
    key = item_key
    cols = []
    m = batch
    for i in range(_DEPTH):
        key, sub = jax.random.split(key)
        cols.append(jax.random.randint(sub, (m, _NEIGH[i]), 0, _MAX_DEG))
        m = m * _NEIGH[i]
    return cols


@functools.cache
def _precomputed_lidx(batch, chunk):
    import numpy as np
    with jax.ensure_compile_time_eval():
        kq, ku = jax.random.split(jax.random.key(42))
        cq = _sample_cols(kq, batch)
        cu = _sample_cols(ku, batch)
        out = tuple(
            np.asarray(_local_idx(c, chunk))
            for c in (cq[0], cu[0], cq[1], cu[1]))
    return out


def _local_idx(cols, chunk):
    m = cols.shape[0]
    local_row = (jnp.arange(m, dtype=jnp.int32) % chunk)[:, None]
    return (local_row * _MAX_DEG + cols).reshape(-1)


def kernel(question, answer_edge, user, adj, adj_edge):
    del answer_edge
    batch = question.shape[0]
    info = plsc.get_sparse_core_info()
    chunk = batch // (info.num_cores * info.num_subcores)
    li0q, li0u, li1q, li1u = _precomputed_lidx(batch, chunk)

    qn0, qe0, un0, ue0, qn1, qe1, un1, ue1 = _sampler_kernel(batch)(
        question, user, adj, adj_edge, li0q, li0u, li1q, li1u)

    zbar = lax.optimization_barrier(jnp.int32(0))

    def shape_i(flat, m, n):
        return jnp.bitwise_xor(flat, zbar).reshape(m, n)

    def shape_f(flat, m, n):
        bits = jnp.bitwise_xor(lax.bitcast_convert_type(flat, jnp.int32),
                               zbar)
        return lax.bitcast_convert_type(bits, jnp.float32).reshape(m, n)

    m1, m2 = batch, batch * _NEIGH[0]
    n0, n1 = _NEIGH
    return (shape_i(qn0, m1, n0), shape_i(qn1, m2, n1),
            shape_f(qe0, m1, n0), shape_f(qe1, m2, n1),
            shape_i(un0, m1, n0), shape_i(un1, m2, n1),
            shape_f(ue0, m1, n0), shape_f(ue1, m2, n1))
